# baseline (speedup 1.0000x reference)
import functools

import jax
import jax.numpy as jnp
from jax.experimental import pallas as pl
from jax.experimental.pallas import tpu as pltpu

_BATCH = 32
_VOCAB = 1_000_000
_CHUNK = 32768

_K0 = 0
_K1 = 42
_KS2 = 0x1BD11BDA ^ _K0 ^ _K1
_TINY = float(jnp.finfo(jnp.float32).tiny)

_ROT = ((13, 15, 26, 6), (17, 29, 16, 24))


def _rotl(x, r):
    return (x << jnp.uint32(r)) | (x >> jnp.uint32(32 - r))


def _threefry_bits(flat):
    ks = (jnp.uint32(_K0), jnp.uint32(_K1), jnp.uint32(_KS2))
    x0 = jnp.full_like(flat, ks[0])
    x1 = flat + ks[1]
    for i in range(5):
        for r in _ROT[i % 2]:
            x0 = x0 + x1
            x1 = _rotl(x1, r) ^ x0
        x0 = x0 + ks[(i + 1) % 3]
        x1 = x1 + ks[(i + 2) % 3] + jnp.uint32(i + 1)
    return x0 ^ x1


def _gumbel_from_bits(bits):
    fb = (bits >> jnp.uint32(9)) | jnp.uint32(0x3F800000)
    f = jax.lax.bitcast_convert_type(fb, jnp.float32) - jnp.float32(1.0)
    u = jnp.maximum(f, jnp.float32(_TINY))
    return -jnp.log(-jnp.log(u))


def _sample_kernel(x_ref, o_ref, acc_val, acc_step, *, nsteps, local_v):
    j = pl.program_id(0)

    @pl.when(j == 0)
    def _init():
        acc_val[...] = jnp.full((_BATCH, _CHUNK), -jnp.inf, jnp.float32)
        acc_step[...] = jnp.zeros((_BATCH, _CHUNK), jnp.int32)

    row = jax.lax.broadcasted_iota(jnp.uint32, (_BATCH, _CHUNK), 0)
    col = jax.lax.broadcasted_iota(jnp.uint32, (_BATCH, _CHUNK), 1)
    flat = row * jnp.uint32(_VOCAB) + col + j.astype(jnp.uint32) * jnp.uint32(_CHUNK)

    g = _gumbel_from_bits(_threefry_bits(flat))
    val = x_ref[...] + g
    lim = local_v - j * _CHUNK
    icol = jax.lax.broadcasted_iota(jnp.int32, (_BATCH, _CHUNK), 1)
    val = jnp.where(icol < lim, val, -jnp.inf)

    take = val > acc_val[...]
    acc_val[...] = jnp.where(take, val, acc_val[...])
    acc_step[...] = jnp.where(take, j, acc_step[...])

    @pl.when(j == nsteps - 1)
    def _finish():
        av = acc_val[...]
        m = jnp.max(av, axis=1, keepdims=True)
        idx = acc_step[...] * _CHUNK + jax.lax.broadcasted_iota(
            jnp.int32, (_BATCH, _CHUNK), 1)
        cand = jnp.where(av == m, idx, jnp.int32(0x7FFFFFFF))
        o_ref[...] = jnp.min(cand, axis=1, keepdims=True)


def kernel(logits):
    nsteps = (_VOCAB + _CHUNK - 1) // _CHUNK
    return pl.pallas_call(
        functools.partial(_sample_kernel, nsteps=nsteps, local_v=_VOCAB),
        grid=(nsteps,),
        in_specs=[pl.BlockSpec((_BATCH, _CHUNK), lambda j: (0, j))],
        out_specs=pl.BlockSpec((_BATCH, 1), lambda j: (0, 0)),
        out_shape=jax.ShapeDtypeStruct((_BATCH, 1), jnp.int32),
        scratch_shapes=[
            pltpu.VMEM((_BATCH, _CHUNK), jnp.float32),
            pltpu.VMEM((_BATCH, _CHUNK), jnp.int32),
        ],
    )(logits)


# speedup vs baseline: 1.0227x; 1.0227x over previous
import functools

import jax
import jax.numpy as jnp
from jax.experimental import pallas as pl
from jax.experimental.pallas import tpu as pltpu

_BATCH = 32
_VOCAB = 1_000_000
_CHUNK = 8192

_K0 = 0
_K1 = 42
_KS2 = 0x1BD11BDA ^ _K0 ^ _K1
_TINY = float(jnp.finfo(jnp.float32).tiny)

_ROT = ((13, 15, 26, 6), (17, 29, 16, 24))


def _rotl(x, r):
    return (x << jnp.uint32(r)) | (x >> jnp.uint32(32 - r))


def _threefry_bits(x1):
    ks = (jnp.uint32(_K0), jnp.uint32(_K1), jnp.uint32(_KS2))
    x0 = jnp.full_like(x1, ks[0])
    for i in range(5):
        for r in _ROT[i % 2]:
            x0 = x0 + x1
            x1 = _rotl(x1, r) ^ x0
        x0 = x0 + ks[(i + 1) % 3]
        x1 = x1 + ks[(i + 2) % 3] + jnp.uint32(i + 1)
    return x0 ^ x1


def _gumbel_from_bits(bits):
    fb = (bits >> jnp.uint32(9)) | jnp.uint32(0x3F800000)
    f = jax.lax.bitcast_convert_type(fb, jnp.float32) - jnp.float32(1.0)
    u = jnp.maximum(f, jnp.float32(_TINY))
    return -jnp.log(-jnp.log(u))


def _sample_kernel(x_ref, o_ref, acc_val, acc_step, x1b, *, nsteps, local_v):
    j = pl.program_id(0)

    @pl.when(j == 0)
    def _init():
        acc_val[...] = jnp.full((_BATCH, _CHUNK), -jnp.inf, jnp.float32)
        acc_step[...] = jnp.zeros((_BATCH, _CHUNK), jnp.int32)
        row = jax.lax.broadcasted_iota(jnp.uint32, (_BATCH, _CHUNK), 0)
        col = jax.lax.broadcasted_iota(jnp.uint32, (_BATCH, _CHUNK), 1)
        x1b[...] = row * jnp.uint32(_VOCAB) + col + jnp.uint32(_K1)

    x1 = x1b[...] + j.astype(jnp.uint32) * jnp.uint32(_CHUNK)

    g = _gumbel_from_bits(_threefry_bits(x1))
    val = x_ref[...] + g
    lim = local_v - j * _CHUNK
    icol = jax.lax.broadcasted_iota(jnp.int32, (_BATCH, _CHUNK), 1)
    val = jnp.where(icol < lim, val, -jnp.inf)

    take = val > acc_val[...]
    acc_val[...] = jnp.where(take, val, acc_val[...])
    acc_step[...] = jnp.where(take, j, acc_step[...])

    @pl.when(j == nsteps - 1)
    def _finish():
        av = acc_val[...]
        m = jnp.max(av, axis=1, keepdims=True)
        idx = acc_step[...] * _CHUNK + jax.lax.broadcasted_iota(
            jnp.int32, (_BATCH, _CHUNK), 1)
        cand = jnp.where(av == m, idx, jnp.int32(0x7FFFFFFF))
        o_ref[...] = jnp.min(cand, axis=1, keepdims=True)


def kernel(logits):
    nsteps = (_VOCAB + _CHUNK - 1) // _CHUNK
    return pl.pallas_call(
        functools.partial(_sample_kernel, nsteps=nsteps, local_v=_VOCAB),
        grid=(nsteps,),
        in_specs=[pl.BlockSpec((_BATCH, _CHUNK), lambda j: (0, j))],
        out_specs=pl.BlockSpec((_BATCH, 1), lambda j: (0, 0)),
        out_shape=jax.ShapeDtypeStruct((_BATCH, 1), jnp.int32),
        scratch_shapes=[
            pltpu.VMEM((_BATCH, _CHUNK), jnp.float32),
            pltpu.VMEM((_BATCH, _CHUNK), jnp.int32),
            pltpu.VMEM((_BATCH, _CHUNK), jnp.uint32),
        ],
    )(logits)
